# baseline (speedup 1.0000x reference)
import functools

import jax
import jax.numpy as jnp
from jax.experimental import pallas as pl

H = 16
EPS = 1e-6
NORM_MIN = 1e-12
TOP_K = 2
E = 8

_PREC = jax.lax.Precision.HIGHEST


def _dot(a, b):
    return jax.lax.dot_general(
        a, b, (((a.ndim - 1,), (0,)), ((), ())),
        preferred_element_type=jnp.float32, precision=_PREC)


def _qkv_kernel(x_ref, w_ref, wq_ref, cos_ref, sin_ref, o_ref, *, dh):
    j = pl.program_id(0)
    x = x_ref[...]
    var = jnp.mean(x * x, axis=-1, keepdims=True)
    normed = x * jax.lax.rsqrt(var + EPS) * w_ref[...]
    y = _dot(normed, wq_ref[0])
    c = cos_ref[...]
    s = sin_ref[...]
    hd = dh // 2
    is_qk = j < 2
    for h in range(H):
        a = y[:, h * dh:h * dh + hd]
        b = y[:, h * dh + hd:(h + 1) * dh]
        ra = jnp.where(is_qk, a * c - b * s, a)
        rb = jnp.where(is_qk, b * c + a * s, b)
        o_ref[0, :, h * dh:h * dh + hd] = ra
        o_ref[0, :, h * dh + hd:(h + 1) * dh] = rb


def _attn_kernel(q_ref, k_ref, v_ref, o_ref, *, qb, dh, s_len):
    i = pl.program_id(1)
    q = q_ref[...]
    k = k_ref[...]
    v = v_ref[...]
    scores = jax.lax.dot_general(
        q, k, (((1,), (1,)), ((), ())),
        preferred_element_type=jnp.float32, precision=_PREC)
    scores = scores * (1.0 / (dh ** 0.5))
    row = i * qb + jax.lax.broadcasted_iota(jnp.int32, (qb, s_len), 0)
    col = jax.lax.broadcasted_iota(jnp.int32, (qb, s_len), 1)
    scores = jnp.where(col <= row, scores, jnp.float32(-1e9))
    m = jnp.max(scores, axis=-1, keepdims=True)
    p = jnp.exp(scores - m)
    p = p / jnp.sum(p, axis=-1, keepdims=True)
    o_ref[...] = _dot(p, v)


def _post_kernel(attn_ref, resid_ref, w2_ref, wo_ref, gw_ref,
                 h1_ref, x2_ref, logits_ref):
    o = _dot(attn_ref[...], wo_ref[...])
    h1 = resid_ref[...] + o
    h1_ref[...] = h1
    var = jnp.mean(h1 * h1, axis=-1, keepdims=True)
    x2 = h1 * jax.lax.rsqrt(var + EPS) * w2_ref[...]
    x2_ref[...] = x2
    logits_ref[...] = _dot(x2, gw_ref[...])


def _router_kernel(logits_ref, bias_ref, we_ref, *, s_len):
    z = logits_ref[...]
    m = jnp.max(z, axis=-1, keepdims=True)
    p = jnp.exp(z - m)
    p = p / jnp.sum(p, axis=-1, keepdims=True)
    corrected = p + bias_ref[...]
    col = jax.lax.broadcasted_iota(jnp.int32, (s_len, E), 1)
    c1 = jnp.max(corrected, axis=-1, keepdims=True)
    i1 = jnp.min(jnp.where(corrected == c1, col, E), axis=-1, keepdims=True)
    masked = jnp.where(col == i1, -jnp.inf, corrected)
    c2 = jnp.max(masked, axis=-1, keepdims=True)
    i2 = jnp.min(jnp.where(masked == c2, col, E), axis=-1, keepdims=True)
    rw1 = jnp.sum(jnp.where(col == i1, p, 0.0), axis=-1, keepdims=True)
    rw2 = jnp.sum(jnp.where(col == i2, p, 0.0), axis=-1, keepdims=True)
    denom = jnp.clip(rw1 + rw2, NORM_MIN, None)
    we = (jnp.where(col == i1, rw1, 0.0) + jnp.where(col == i2, rw2, 0.0)) / denom
    we_ref[...] = we


def _moe_kernel(x_ref, h1_ref, we_ref, wg_ref, wu_ref, wd_ref, o_ref, *, sb):
    e = pl.program_id(0)
    f = pl.program_id(1)
    s = pl.program_id(2)
    x = x_ref[...]
    g = _dot(x, wg_ref[0])
    u = _dot(x, wu_ref[0])
    hh = (g * jax.lax.logistic(g)) * u
    y = _dot(hh, wd_ref[0])
    we = we_ref[0, 0, :][:, None]
    contrib = y * we
    rows = pl.ds(s * sb, sb)

    @pl.when(jnp.logical_and(e == 0, f == 0))
    def _():
        o_ref[rows, :] = h1_ref[...] + contrib

    @pl.when(jnp.logical_not(jnp.logical_and(e == 0, f == 0)))
    def _():
        o_ref[rows, :] = o_ref[rows, :] + contrib


def kernel(hidden_states, ln1_w, ln2_w, Wq, Wk, Wv, Wo, gate_w, bias, cos, sin,
           Wg, Wu, Wd):
    B, S, D = hidden_states.shape
    dh = D // H
    Dff = Wg.shape[-1]
    xf = hidden_states.reshape(S, D)

    def _deinterleave_cols(W):
        return W.reshape(D, H, dh // 2, 2).transpose(0, 1, 3, 2).reshape(D, D)

    Wqkv = jnp.stack([_deinterleave_cols(Wq), _deinterleave_cols(Wk), Wv])
    cos_h = cos[:, 0::2]
    sin_h = sin[:, 0::2]
    ln1 = ln1_w.reshape(1, D)
    ln2 = ln2_w.reshape(1, D)
    bias2 = bias.reshape(1, E)

    SB = 256
    n_s = S // SB

    qkv = pl.pallas_call(
        functools.partial(_qkv_kernel, dh=dh),
        grid=(3, n_s),
        in_specs=[
            pl.BlockSpec((SB, D), lambda j, s: (s, 0)),
            pl.BlockSpec((1, D), lambda j, s: (0, 0)),
            pl.BlockSpec((1, D, D), lambda j, s: (j, 0, 0)),
            pl.BlockSpec((SB, dh // 2), lambda j, s: (s, 0)),
            pl.BlockSpec((SB, dh // 2), lambda j, s: (s, 0)),
        ],
        out_specs=pl.BlockSpec((1, SB, D), lambda j, s: (j, s, 0)),
        out_shape=jax.ShapeDtypeStruct((3, S, D), jnp.float32),
    )(xf, ln1, Wqkv, cos_h, sin_h)
    q, k, v = qkv[0], qkv[1], qkv[2]

    QB = 256
    attn = pl.pallas_call(
        functools.partial(_attn_kernel, qb=QB, dh=dh, s_len=S),
        grid=(H, S // QB),
        in_specs=[
            pl.BlockSpec((QB, dh), lambda h, i: (i, h)),
            pl.BlockSpec((S, dh), lambda h, i: (0, h)),
            pl.BlockSpec((S, dh), lambda h, i: (0, h)),
        ],
        out_specs=pl.BlockSpec((QB, dh), lambda h, i: (i, h)),
        out_shape=jax.ShapeDtypeStruct((S, D), jnp.float32),
    )(q, k, v)

    h1, x2, logits = pl.pallas_call(
        _post_kernel,
        grid=(n_s,),
        in_specs=[
            pl.BlockSpec((SB, D), lambda s: (s, 0)),
            pl.BlockSpec((SB, D), lambda s: (s, 0)),
            pl.BlockSpec((1, D), lambda s: (0, 0)),
            pl.BlockSpec((D, D), lambda s: (0, 0)),
            pl.BlockSpec((D, E), lambda s: (0, 0)),
        ],
        out_specs=[
            pl.BlockSpec((SB, D), lambda s: (s, 0)),
            pl.BlockSpec((SB, D), lambda s: (s, 0)),
            pl.BlockSpec((SB, E), lambda s: (s, 0)),
        ],
        out_shape=[
            jax.ShapeDtypeStruct((S, D), jnp.float32),
            jax.ShapeDtypeStruct((S, D), jnp.float32),
            jax.ShapeDtypeStruct((S, E), jnp.float32),
        ],
    )(attn, xf, ln2, Wo, gate_w)

    we8 = pl.pallas_call(
        functools.partial(_router_kernel, s_len=S),
        grid=(1,),
        in_specs=[
            pl.BlockSpec((S, E), lambda i: (0, 0)),
            pl.BlockSpec((1, E), lambda i: (0, 0)),
        ],
        out_specs=pl.BlockSpec((S, E), lambda i: (0, 0)),
        out_shape=jax.ShapeDtypeStruct((S, E), jnp.float32),
    )(logits, bias2)
    weT = we8.T.reshape(E, 1, S)

    FB = 512
    n_f = Dff // FB
    out = pl.pallas_call(
        functools.partial(_moe_kernel, sb=SB),
        grid=(E, n_f, n_s),
        in_specs=[
            pl.BlockSpec((SB, D), lambda e, f, s: (s, 0)),
            pl.BlockSpec((SB, D), lambda e, f, s: (s, 0)),
            pl.BlockSpec((1, 1, SB), lambda e, f, s: (e, 0, s)),
            pl.BlockSpec((1, D, FB), lambda e, f, s: (e, 0, f)),
            pl.BlockSpec((1, D, FB), lambda e, f, s: (e, 0, f)),
            pl.BlockSpec((1, FB, D), lambda e, f, s: (e, f, 0)),
        ],
        out_specs=pl.BlockSpec((S, D), lambda e, f, s: (0, 0)),
        out_shape=jax.ShapeDtypeStruct((S, D), jnp.float32),
    )(x2, h1, weT, Wg, Wu, Wd)

    return out.reshape(B, S, D)


# speedup vs baseline: 3.4057x; 3.4057x over previous
import functools

import jax
import jax.numpy as jnp
from jax.experimental import pallas as pl

H = 16
EPS = 1e-6
NORM_MIN = 1e-12
TOP_K = 2
E = 8

_PREC = jax.lax.Precision.DEFAULT


def _dot(a, b):
    return jax.lax.dot_general(
        a.astype(jnp.bfloat16), b.astype(jnp.bfloat16),
        (((a.ndim - 1,), (0,)), ((), ())),
        preferred_element_type=jnp.float32, precision=_PREC)


def _qkv_kernel(x_ref, w_ref, wq_ref, cos_ref, sin_ref, o_ref, *, dh):
    j = pl.program_id(0)
    x = x_ref[...]
    var = jnp.mean(x * x, axis=-1, keepdims=True)
    normed = x * jax.lax.rsqrt(var + EPS) * w_ref[...]
    y = _dot(normed, wq_ref[0])
    c = cos_ref[...]
    s = sin_ref[...]
    hd = dh // 2
    is_qk = j < 2
    for h in range(H):
        a = y[:, h * dh:h * dh + hd]
        b = y[:, h * dh + hd:(h + 1) * dh]
        ra = jnp.where(is_qk, a * c - b * s, a)
        rb = jnp.where(is_qk, b * c + a * s, b)
        o_ref[0, :, h * dh:h * dh + hd] = ra
        o_ref[0, :, h * dh + hd:(h + 1) * dh] = rb


def _attn_kernel(q_ref, k_ref, v_ref, o_ref, *, qb, dh, s_len):
    i = pl.program_id(1)
    q = q_ref[...]
    k = k_ref[...]
    v = v_ref[...]
    scores = jax.lax.dot_general(
        q.astype(jnp.bfloat16), k.astype(jnp.bfloat16),
        (((1,), (1,)), ((), ())),
        preferred_element_type=jnp.float32, precision=_PREC)
    scores = scores * (1.0 / (dh ** 0.5))
    row = i * qb + jax.lax.broadcasted_iota(jnp.int32, (qb, s_len), 0)
    col = jax.lax.broadcasted_iota(jnp.int32, (qb, s_len), 1)
    scores = jnp.where(col <= row, scores, jnp.float32(-1e9))
    m = jnp.max(scores, axis=-1, keepdims=True)
    p = jnp.exp(scores - m)
    p = p / jnp.sum(p, axis=-1, keepdims=True)
    o_ref[...] = _dot(p, v)


def _post_kernel(attn_ref, resid_ref, w2_ref, wo_ref, gw_ref,
                 h1_ref, x2_ref, logits_ref):
    o = _dot(attn_ref[...], wo_ref[...])
    h1 = resid_ref[...] + o
    h1_ref[...] = h1
    var = jnp.mean(h1 * h1, axis=-1, keepdims=True)
    x2 = h1 * jax.lax.rsqrt(var + EPS) * w2_ref[...]
    x2_ref[...] = x2
    logits_ref[...] = _dot(x2, gw_ref[...])


def _router_kernel(logits_ref, bias_ref, we_ref, *, s_len):
    z = logits_ref[...]
    m = jnp.max(z, axis=-1, keepdims=True)
    p = jnp.exp(z - m)
    p = p / jnp.sum(p, axis=-1, keepdims=True)
    corrected = p + bias_ref[...]
    col = jax.lax.broadcasted_iota(jnp.int32, (s_len, E), 1)
    c1 = jnp.max(corrected, axis=-1, keepdims=True)
    i1 = jnp.min(jnp.where(corrected == c1, col, E), axis=-1, keepdims=True)
    masked = jnp.where(col == i1, -jnp.inf, corrected)
    c2 = jnp.max(masked, axis=-1, keepdims=True)
    i2 = jnp.min(jnp.where(masked == c2, col, E), axis=-1, keepdims=True)
    rw1 = jnp.sum(jnp.where(col == i1, p, 0.0), axis=-1, keepdims=True)
    rw2 = jnp.sum(jnp.where(col == i2, p, 0.0), axis=-1, keepdims=True)
    denom = jnp.clip(rw1 + rw2, NORM_MIN, None)
    we = (jnp.where(col == i1, rw1, 0.0) + jnp.where(col == i2, rw2, 0.0)) / denom
    we_ref[...] = we


def _moe_kernel(x_ref, h1_ref, we_ref, wg_ref, wu_ref, wd_ref, o_ref, *, sb):
    e = pl.program_id(0)
    f = pl.program_id(1)
    s = pl.program_id(2)
    x = x_ref[...]
    g = _dot(x, wg_ref[0])
    u = _dot(x, wu_ref[0])
    hh = (g * jax.lax.logistic(g)) * u
    y = _dot(hh, wd_ref[0])
    we = we_ref[0, 0, :][:, None]
    contrib = y * we
    rows = pl.ds(s * sb, sb)

    @pl.when(jnp.logical_and(e == 0, f == 0))
    def _():
        o_ref[rows, :] = h1_ref[...] + contrib

    @pl.when(jnp.logical_not(jnp.logical_and(e == 0, f == 0)))
    def _():
        o_ref[rows, :] = o_ref[rows, :] + contrib


def kernel(hidden_states, ln1_w, ln2_w, Wq, Wk, Wv, Wo, gate_w, bias, cos, sin,
           Wg, Wu, Wd):
    B, S, D = hidden_states.shape
    dh = D // H
    Dff = Wg.shape[-1]
    xf = hidden_states.reshape(S, D)

    def _deinterleave_cols(W):
        return W.reshape(D, H, dh // 2, 2).transpose(0, 1, 3, 2).reshape(D, D)

    Wqkv = jnp.stack([_deinterleave_cols(Wq), _deinterleave_cols(Wk), Wv])
    cos_h = cos[:, 0::2]
    sin_h = sin[:, 0::2]
    ln1 = ln1_w.reshape(1, D)
    ln2 = ln2_w.reshape(1, D)
    bias2 = bias.reshape(1, E)

    SB = 256
    n_s = S // SB

    qkv = pl.pallas_call(
        functools.partial(_qkv_kernel, dh=dh),
        grid=(3, n_s),
        in_specs=[
            pl.BlockSpec((SB, D), lambda j, s: (s, 0)),
            pl.BlockSpec((1, D), lambda j, s: (0, 0)),
            pl.BlockSpec((1, D, D), lambda j, s: (j, 0, 0)),
            pl.BlockSpec((SB, dh // 2), lambda j, s: (s, 0)),
            pl.BlockSpec((SB, dh // 2), lambda j, s: (s, 0)),
        ],
        out_specs=pl.BlockSpec((1, SB, D), lambda j, s: (j, s, 0)),
        out_shape=jax.ShapeDtypeStruct((3, S, D), jnp.float32),
    )(xf, ln1, Wqkv, cos_h, sin_h)
    q, k, v = qkv[0], qkv[1], qkv[2]

    QB = 256
    attn = pl.pallas_call(
        functools.partial(_attn_kernel, qb=QB, dh=dh, s_len=S),
        grid=(H, S // QB),
        in_specs=[
            pl.BlockSpec((QB, dh), lambda h, i: (i, h)),
            pl.BlockSpec((S, dh), lambda h, i: (0, h)),
            pl.BlockSpec((S, dh), lambda h, i: (0, h)),
        ],
        out_specs=pl.BlockSpec((QB, dh), lambda h, i: (i, h)),
        out_shape=jax.ShapeDtypeStruct((S, D), jnp.float32),
    )(q, k, v)

    h1, x2, logits = pl.pallas_call(
        _post_kernel,
        grid=(n_s,),
        in_specs=[
            pl.BlockSpec((SB, D), lambda s: (s, 0)),
            pl.BlockSpec((SB, D), lambda s: (s, 0)),
            pl.BlockSpec((1, D), lambda s: (0, 0)),
            pl.BlockSpec((D, D), lambda s: (0, 0)),
            pl.BlockSpec((D, E), lambda s: (0, 0)),
        ],
        out_specs=[
            pl.BlockSpec((SB, D), lambda s: (s, 0)),
            pl.BlockSpec((SB, D), lambda s: (s, 0)),
            pl.BlockSpec((SB, E), lambda s: (s, 0)),
        ],
        out_shape=[
            jax.ShapeDtypeStruct((S, D), jnp.float32),
            jax.ShapeDtypeStruct((S, D), jnp.float32),
            jax.ShapeDtypeStruct((S, E), jnp.float32),
        ],
    )(attn, xf, ln2, Wo, gate_w)

    we8 = pl.pallas_call(
        functools.partial(_router_kernel, s_len=S),
        grid=(1,),
        in_specs=[
            pl.BlockSpec((S, E), lambda i: (0, 0)),
            pl.BlockSpec((1, E), lambda i: (0, 0)),
        ],
        out_specs=pl.BlockSpec((S, E), lambda i: (0, 0)),
        out_shape=jax.ShapeDtypeStruct((S, E), jnp.float32),
    )(logits, bias2)
    weT = we8.T.reshape(E, 1, S)

    FB = 512
    n_f = Dff // FB
    out = pl.pallas_call(
        functools.partial(_moe_kernel, sb=SB),
        grid=(E, n_f, n_s),
        in_specs=[
            pl.BlockSpec((SB, D), lambda e, f, s: (s, 0)),
            pl.BlockSpec((SB, D), lambda e, f, s: (s, 0)),
            pl.BlockSpec((1, 1, SB), lambda e, f, s: (e, 0, s)),
            pl.BlockSpec((1, D, FB), lambda e, f, s: (e, 0, f)),
            pl.BlockSpec((1, D, FB), lambda e, f, s: (e, 0, f)),
            pl.BlockSpec((1, FB, D), lambda e, f, s: (e, f, 0)),
        ],
        out_specs=pl.BlockSpec((S, D), lambda e, f, s: (0, 0)),
        out_shape=jax.ShapeDtypeStruct((S, D), jnp.float32),
    )(x2, h1, weT, Wg, Wu, Wd)

    return out.reshape(B, S, D)


# speedup vs baseline: 3.5181x; 1.0330x over previous
import functools

import jax
import jax.numpy as jnp
from jax.experimental import pallas as pl

H = 16
EPS = 1e-6
NORM_MIN = 1e-12
TOP_K = 2
E = 8

_PREC = jax.lax.Precision.DEFAULT


def _dot(a, b):
    return jax.lax.dot_general(
        a.astype(jnp.bfloat16), b.astype(jnp.bfloat16),
        (((a.ndim - 1,), (0,)), ((), ())),
        preferred_element_type=jnp.float32, precision=_PREC)


def _qkv_kernel(x_ref, w_ref, wq_ref, cos_ref, sin_ref, o_ref, *, dh):
    j = pl.program_id(0)
    x = x_ref[...]
    var = jnp.mean(x * x, axis=-1, keepdims=True)
    normed = x * jax.lax.rsqrt(var + EPS) * w_ref[...]
    y = _dot(normed, wq_ref[0])
    c = cos_ref[...]
    s = sin_ref[...]
    hd = dh // 2
    is_qk = j < 2
    for h in range(H):
        a = y[:, h * dh:h * dh + hd]
        b = y[:, h * dh + hd:(h + 1) * dh]
        ra = jnp.where(is_qk, a * c - b * s, a)
        rb = jnp.where(is_qk, b * c + a * s, b)
        o_ref[0, :, h * dh:h * dh + hd] = ra
        o_ref[0, :, h * dh + hd:(h + 1) * dh] = rb


def _attn_kernel(q_ref, k_ref, v_ref, o_ref, *, qb, dh, s_len):
    i = pl.program_id(1)
    q = q_ref[0]
    k = k_ref[0]
    v = v_ref[0]
    scores = jax.lax.dot_general(
        q.astype(jnp.bfloat16), k.astype(jnp.bfloat16),
        (((1,), (1,)), ((), ())),
        preferred_element_type=jnp.float32, precision=_PREC)
    scores = scores * (1.0 / (dh ** 0.5))
    row = i * qb + jax.lax.broadcasted_iota(jnp.int32, (qb, s_len), 0)
    col = jax.lax.broadcasted_iota(jnp.int32, (qb, s_len), 1)
    scores = jnp.where(col <= row, scores, jnp.float32(-1e9))
    m = jnp.max(scores, axis=-1, keepdims=True)
    p = jnp.exp(scores - m)
    p = p / jnp.sum(p, axis=-1, keepdims=True)
    o_ref[...] = _dot(p, v)


def _post_kernel(attn_ref, resid_ref, w2_ref, wo_ref, gw_ref,
                 h1_ref, x2_ref, logits_ref):
    o = _dot(attn_ref[...], wo_ref[...])
    h1 = resid_ref[...] + o
    h1_ref[...] = h1
    var = jnp.mean(h1 * h1, axis=-1, keepdims=True)
    x2 = h1 * jax.lax.rsqrt(var + EPS) * w2_ref[...]
    x2_ref[...] = x2
    logits_ref[...] = _dot(x2, gw_ref[...])


def _router_kernel(logits_ref, bias_ref, we_ref, *, s_len):
    z = logits_ref[...]
    m = jnp.max(z, axis=-1, keepdims=True)
    p = jnp.exp(z - m)
    p = p / jnp.sum(p, axis=-1, keepdims=True)
    corrected = p + bias_ref[...]
    col = jax.lax.broadcasted_iota(jnp.int32, (s_len, E), 1)
    c1 = jnp.max(corrected, axis=-1, keepdims=True)
    i1 = jnp.min(jnp.where(corrected == c1, col, E), axis=-1, keepdims=True)
    masked = jnp.where(col == i1, -jnp.inf, corrected)
    c2 = jnp.max(masked, axis=-1, keepdims=True)
    i2 = jnp.min(jnp.where(masked == c2, col, E), axis=-1, keepdims=True)
    rw1 = jnp.sum(jnp.where(col == i1, p, 0.0), axis=-1, keepdims=True)
    rw2 = jnp.sum(jnp.where(col == i2, p, 0.0), axis=-1, keepdims=True)
    denom = jnp.clip(rw1 + rw2, NORM_MIN, None)
    we = (jnp.where(col == i1, rw1, 0.0) + jnp.where(col == i2, rw2, 0.0)) / denom
    we_ref[...] = we.T.reshape(E, 1, s_len)


def _moe_kernel(x_ref, h1_ref, we_ref, wg_ref, wu_ref, wd_ref, o_ref, *, sb):
    e = pl.program_id(0)
    f = pl.program_id(1)
    s = pl.program_id(2)
    x = x_ref[...]
    g = _dot(x, wg_ref[0])
    u = _dot(x, wu_ref[0])
    hh = (g * jax.lax.logistic(g)) * u
    y = _dot(hh, wd_ref[0])
    we = we_ref[0, 0, :][:, None]
    contrib = y * we
    rows = pl.ds(s * sb, sb)

    @pl.when(jnp.logical_and(e == 0, f == 0))
    def _():
        o_ref[rows, :] = h1_ref[...] + contrib

    @pl.when(jnp.logical_not(jnp.logical_and(e == 0, f == 0)))
    def _():
        o_ref[rows, :] = o_ref[rows, :] + contrib


def kernel(hidden_states, ln1_w, ln2_w, Wq, Wk, Wv, Wo, gate_w, bias, cos, sin,
           Wg, Wu, Wd):
    B, S, D = hidden_states.shape
    dh = D // H
    Dff = Wg.shape[-1]
    xf = hidden_states.reshape(S, D)

    def _deinterleave_cols(W):
        return W.reshape(D, H, dh // 2, 2).transpose(0, 1, 3, 2).reshape(D, D)

    Wqkv = jnp.stack([_deinterleave_cols(Wq), _deinterleave_cols(Wk), Wv])
    cos_h = cos[:, 0::2]
    sin_h = sin[:, 0::2]
    ln1 = ln1_w.reshape(1, D)
    ln2 = ln2_w.reshape(1, D)
    bias2 = bias.reshape(1, E)

    SB = 256
    n_s = S // SB

    qkv = pl.pallas_call(
        functools.partial(_qkv_kernel, dh=dh),
        grid=(3, n_s),
        in_specs=[
            pl.BlockSpec((SB, D), lambda j, s: (s, 0)),
            pl.BlockSpec((1, D), lambda j, s: (0, 0)),
            pl.BlockSpec((1, D, D), lambda j, s: (j, 0, 0)),
            pl.BlockSpec((SB, dh // 2), lambda j, s: (s, 0)),
            pl.BlockSpec((SB, dh // 2), lambda j, s: (s, 0)),
        ],
        out_specs=pl.BlockSpec((1, SB, D), lambda j, s: (j, s, 0)),
        out_shape=jax.ShapeDtypeStruct((3, S, D), jnp.float32),
    )(xf, ln1, Wqkv, cos_h, sin_h)

    QB = 256
    attn = pl.pallas_call(
        functools.partial(_attn_kernel, qb=QB, dh=dh, s_len=S),
        grid=(H, S // QB),
        in_specs=[
            pl.BlockSpec((1, QB, dh), lambda h, i: (0, i, h)),
            pl.BlockSpec((1, S, dh), lambda h, i: (1, 0, h)),
            pl.BlockSpec((1, S, dh), lambda h, i: (2, 0, h)),
        ],
        out_specs=pl.BlockSpec((QB, dh), lambda h, i: (i, h)),
        out_shape=jax.ShapeDtypeStruct((S, D), jnp.float32),
    )(qkv, qkv, qkv)

    h1, x2, logits = pl.pallas_call(
        _post_kernel,
        grid=(n_s,),
        in_specs=[
            pl.BlockSpec((SB, D), lambda s: (s, 0)),
            pl.BlockSpec((SB, D), lambda s: (s, 0)),
            pl.BlockSpec((1, D), lambda s: (0, 0)),
            pl.BlockSpec((D, D), lambda s: (0, 0)),
            pl.BlockSpec((D, E), lambda s: (0, 0)),
        ],
        out_specs=[
            pl.BlockSpec((SB, D), lambda s: (s, 0)),
            pl.BlockSpec((SB, D), lambda s: (s, 0)),
            pl.BlockSpec((SB, E), lambda s: (s, 0)),
        ],
        out_shape=[
            jax.ShapeDtypeStruct((S, D), jnp.float32),
            jax.ShapeDtypeStruct((S, D), jnp.float32),
            jax.ShapeDtypeStruct((S, E), jnp.float32),
        ],
    )(attn, xf, ln2, Wo, gate_w)

    weT = pl.pallas_call(
        functools.partial(_router_kernel, s_len=S),
        grid=(1,),
        in_specs=[
            pl.BlockSpec((S, E), lambda i: (0, 0)),
            pl.BlockSpec((1, E), lambda i: (0, 0)),
        ],
        out_specs=pl.BlockSpec((E, 1, S), lambda i: (0, 0, 0)),
        out_shape=jax.ShapeDtypeStruct((E, 1, S), jnp.float32),
    )(logits, bias2)

    FB = 512
    n_f = Dff // FB
    out = pl.pallas_call(
        functools.partial(_moe_kernel, sb=SB),
        grid=(E, n_f, n_s),
        in_specs=[
            pl.BlockSpec((SB, D), lambda e, f, s: (s, 0)),
            pl.BlockSpec((SB, D), lambda e, f, s: (s, 0)),
            pl.BlockSpec((1, 1, SB), lambda e, f, s: (e, 0, s)),
            pl.BlockSpec((1, D, FB), lambda e, f, s: (e, 0, f)),
            pl.BlockSpec((1, D, FB), lambda e, f, s: (e, 0, f)),
            pl.BlockSpec((1, FB, D), lambda e, f, s: (e, f, 0)),
        ],
        out_specs=pl.BlockSpec((S, D), lambda e, f, s: (0, 0)),
        out_shape=jax.ShapeDtypeStruct((S, D), jnp.float32),
    )(x2, h1, weT, Wg, Wu, Wd)

    return out.reshape(B, S, D)
